# baseline (speedup 1.0000x reference)
import jax
import jax.numpy as jnp
from jax import lax
from jax.experimental import pallas as pl
from jax.experimental.pallas import tpu as pltpu

_N = 20000
_ROWS = 160
_LANES = 128
_PAD = _ROWS * _LANES
_K = 1000
_DETS = 100
_SCORE_THRESH = 0.05
_NMS_THRESH = 0.5
_NEG_INF = -1e9
_LO_BITS = 0x3D4CCCCD
_HI_BITS = 0x41000000


def _roi_body(
    s_ref, cx_ref, cy_ref, w_ref, h_ref, out_ref,
    x1_ref, y1_ref, x2_ref, y2_ref, ar_ref,
):
    raw = s_ref[...]
    valid = raw > _SCORE_THRESH
    sbits = jnp.where(valid, lax.bitcast_convert_type(raw, jnp.int32), 0)

    cx = cx_ref[...] * 1024.0
    cy = cy_ref[...] * 1024.0
    bw = w_ref[...] * 256.0 + 1.0
    bh = h_ref[...] * 256.0 + 1.0
    x1 = cx - bw * 0.5
    y1 = cy - bh * 0.5
    x2 = cx + bw * 0.5
    y2 = cy + bh * 0.5
    area = jnp.maximum(x2 - x1, 0.0) * jnp.maximum(y2 - y1, 0.0)

    def bs_body(_, carry):
        lo, hi = carry
        mid = (lo + hi) // 2
        c = jnp.sum(jnp.where(sbits >= mid, 1, 0).astype(jnp.int32))
        big = c >= _K
        return jnp.where(big, mid, lo), jnp.where(big, hi, mid)

    lo, _ = lax.fori_loop(
        0, 28, bs_body, (jnp.int32(_LO_BITS), jnp.int32(_HI_BITS))
    )
    x1_ref[...] = x1
    y1_ref[...] = y1
    x2_ref[...] = x2
    y2_ref[...] = y2
    ar_ref[...] = area

    idx = (
        lax.broadcasted_iota(jnp.int32, (_ROWS, _LANES), 0) * _LANES
        + lax.broadcasted_iota(jnp.int32, (_ROWS, _LANES), 1)
    )
    lane = lax.broadcasted_iota(jnp.int32, (1, _LANES), 1)

    def _pluck(ref, r0, lmask):
        return jnp.sum(jnp.where(lmask, ref[pl.ds(r0, 1), :], 0.0))

    def round_body(r, sa):
        m = jnp.max(sa)
        pick = jnp.min(jnp.where(sa == m, idx, jnp.int32(0x7FFFFFFF)))
        r0 = pick // _LANES
        lmask = lane == (pick % _LANES)
        px1 = _pluck(x1_ref, r0, lmask)
        py1 = _pluck(y1_ref, r0, lmask)
        px2 = _pluck(x2_ref, r0, lmask)
        py2 = _pluck(y2_ref, r0, lmask)
        pa = _pluck(ar_ref, r0, lmask)
        iw = jnp.maximum(jnp.minimum(px2, x2) - jnp.maximum(px1, x1), 0.0)
        ih = jnp.maximum(jnp.minimum(py2, y2) - jnp.maximum(py1, y1), 0.0)
        inter = iw * ih
        union = pa + area - inter
        iou = inter / jnp.maximum(union, 1e-9)
        sup = (iou > _NMS_THRESH) | (idx == pick)
        row = jnp.where(
            lane == 0,
            px1,
            jnp.where(
                lane == 1,
                py1,
                jnp.where(
                    lane == 2,
                    px2,
                    jnp.where(lane == 3, py2, jnp.where(lane == 4, m, 0.0)),
                ),
            ),
        )
        out_ref[pl.ds(r, 1), :] = row
        return jnp.where(sup, _NEG_INF, sa)

    lax.fori_loop(0, _DETS, round_body, jnp.where(sbits >= lo, raw, _NEG_INF))


def kernel(boxes, scores):
    pad = _PAD - _N
    s = jnp.pad(scores, (0, pad), constant_values=-1.0).reshape(_ROWS, _LANES)
    cols = []
    for c in range(4):
        col = jnp.pad(boxes[:, c], (0, pad)).reshape(_ROWS, _LANES)
        cols.append(col)
    out = pl.pallas_call(
        _roi_body,
        out_shape=jax.ShapeDtypeStruct((_LANES, _LANES), jnp.float32),
        scratch_shapes=[pltpu.VMEM((_ROWS, _LANES), jnp.float32)] * 5,
    )(s, cols[0], cols[1], cols[2], cols[3])
    return out[:_DETS, :5]


# speedup vs baseline: 1.0695x; 1.0695x over previous
import jax
import jax.numpy as jnp
from jax import lax
from jax.experimental import pallas as pl
from jax.experimental.pallas import tpu as pltpu

_N = 20000
_ROWS = 160
_LANES = 128
_PAD = _ROWS * _LANES
_K = 1000
_DETS = 100
_SCORE_THRESH = 0.05
_NMS_THRESH = 0.5
_NEG_INF = -1e9
_LO_BITS = 0x3D4CCCCD
_HI_BITS = 0x41000000


def _roi_body(
    s_ref, cx_ref, cy_ref, w_ref, h_ref, out_ref,
    x1_ref, y1_ref, x2_ref, y2_ref, ar_ref,
):
    raw = s_ref[...]
    valid = raw > _SCORE_THRESH
    sbits = jnp.where(valid, lax.bitcast_convert_type(raw, jnp.int32), 0)

    cx = cx_ref[...] * 1024.0
    cy = cy_ref[...] * 1024.0
    bw = w_ref[...] * 256.0 + 1.0
    bh = h_ref[...] * 256.0 + 1.0
    x1 = cx - bw * 0.5
    y1 = cy - bh * 0.5
    x2 = cx + bw * 0.5
    y2 = cy + bh * 0.5
    area = jnp.maximum(x2 - x1, 0.0) * jnp.maximum(y2 - y1, 0.0)

    def bs_body(_, carry):
        lo, hi = carry
        mid = (lo + hi) // 2
        c = jnp.sum(jnp.where(sbits >= mid, 1, 0).astype(jnp.int32))
        big = c >= _K
        return jnp.where(big, mid, lo), jnp.where(big, hi, mid)

    lo, _ = lax.fori_loop(
        0, 28, bs_body, (jnp.int32(_LO_BITS), jnp.int32(_HI_BITS))
    )
    x1_ref[...] = x1
    y1_ref[...] = y1
    x2_ref[...] = x2
    y2_ref[...] = y2
    ar_ref[...] = area

    idx = (
        lax.broadcasted_iota(jnp.int32, (_ROWS, _LANES), 0) * _LANES
        + lax.broadcasted_iota(jnp.int32, (_ROWS, _LANES), 1)
    )
    lane = lax.broadcasted_iota(jnp.int32, (1, _LANES), 1)

    def _pluck(ref, r0, lmask):
        return jnp.sum(jnp.where(lmask, ref[pl.ds(r0, 1), :], 0.0))

    def _argpick(sa):
        m = jnp.max(sa)
        pick = jnp.min(jnp.where(sa == m, idx, jnp.int32(0x7FFFFFFF)))
        r0 = pick // _LANES
        lmask = lane == (pick % _LANES)
        return (
            m,
            pick,
            _pluck(x1_ref, r0, lmask),
            _pluck(y1_ref, r0, lmask),
            _pluck(x2_ref, r0, lmask),
            _pluck(y2_ref, r0, lmask),
            _pluck(ar_ref, r0, lmask),
        )

    def _supmask(px1, py1, px2, py2, pa):
        iw = jnp.maximum(jnp.minimum(px2, x2) - jnp.maximum(px1, x1), 0.0)
        ih = jnp.maximum(jnp.minimum(py2, y2) - jnp.maximum(py1, y1), 0.0)
        inter = iw * ih
        union = pa + area - inter
        return inter / jnp.maximum(union, 1e-9) > _NMS_THRESH

    def _mkrow(px1, py1, px2, py2, m):
        return jnp.where(
            lane == 0,
            px1,
            jnp.where(
                lane == 1,
                py1,
                jnp.where(
                    lane == 2,
                    px2,
                    jnp.where(lane == 3, py2, jnp.where(lane == 4, m, 0.0)),
                ),
            ),
        )

    def cond_fn(carry):
        return carry[0] < _DETS

    def body_fn(carry):
        cnt, sa = carry
        m1, pick1, ax1, ay1, ax2, ay2, aa = _argpick(sa)
        sae = jnp.where(idx == pick1, _NEG_INF, sa)
        m2, pick2, bx1, by1, bx2, by2, ba = _argpick(sae)
        siw = jnp.maximum(jnp.minimum(ax2, bx2) - jnp.maximum(ax1, bx1), 0.0)
        sih = jnp.maximum(jnp.minimum(ay2, by2) - jnp.maximum(ay1, by1), 0.0)
        sinter = siw * sih
        sunion = aa + ba - sinter
        ok = jnp.logical_not(
            sinter / jnp.maximum(sunion, 1e-9) > _NMS_THRESH
        )
        sup1 = _supmask(ax1, ay1, ax2, ay2, aa) | (idx == pick1)
        sup2 = _supmask(bx1, by1, bx2, by2, ba) | (idx == pick2)
        out_ref[pl.ds(cnt, 1), :] = _mkrow(ax1, ay1, ax2, ay2, m1)

        @pl.when(ok)
        def _second():
            out_ref[pl.ds(cnt + 1, 1), :] = _mkrow(bx1, by1, bx2, by2, m2)

        sa = jnp.where(sup1 | (ok & sup2), _NEG_INF, sa)
        return cnt + jnp.where(ok, jnp.int32(2), jnp.int32(1)), sa

    lax.while_loop(
        cond_fn, body_fn, (jnp.int32(0), jnp.where(sbits >= lo, raw, _NEG_INF))
    )


def kernel(boxes, scores):
    pad = _PAD - _N
    s = jnp.pad(scores, (0, pad), constant_values=-1.0).reshape(_ROWS, _LANES)
    cols = []
    for c in range(4):
        col = jnp.pad(boxes[:, c], (0, pad)).reshape(_ROWS, _LANES)
        cols.append(col)
    out = pl.pallas_call(
        _roi_body,
        out_shape=jax.ShapeDtypeStruct((_LANES, _LANES), jnp.float32),
        scratch_shapes=[pltpu.VMEM((_ROWS, _LANES), jnp.float32)] * 5,
    )(s, cols[0], cols[1], cols[2], cols[3])
    return out[:_DETS, :5]
